# Initial kernel scaffold; baseline (speedup 1.0000x reference)
import jax
import jax.numpy as jnp
from jax.experimental import pallas as pl


def kernel(input, table, gamma, beta):
    raise NotImplementedError("write your pallas kernel here")


# speedup vs baseline: 42.8161x; 42.8161x over previous
import functools

import jax
import jax.numpy as jnp
from jax import lax
from jax.experimental import pallas as pl
from jax.experimental.pallas import tpu as pltpu
from jax.experimental.pallas import tpu_sc as plsc

NC, NS, LANES = 2, 16, 16
NW = NC * NS
B, L, E = 16384, 100, 64
EC = E // LANES
ROWS_PER_W = B // NW
CHUNK = 8
NSTEPS = ROWS_PER_W // CHUNK
NBUF = 2


def _lane_sum(x):
    lanes = lax.iota(jnp.int32, LANES)
    dnums = lax.GatherDimensionNumbers(
        offset_dims=(), collapsed_slice_dims=(0,), start_index_map=(0,)
    )
    for sh in (1, 2, 4, 8):
        perm = lax.reshape(lanes ^ sh, (LANES, 1))
        x = x + lax.gather(x, perm, dnums, slice_sizes=(1,),
                           mode=lax.GatherScatterMode.PROMISE_IN_BOUNDS)
    return x


def _rsqrt(v):
    i = lax.bitcast_convert_type(v, jnp.int32)
    i = 0x5F3759DF - lax.shift_right_logical(i, 1)
    y = lax.bitcast_convert_type(i, jnp.float32)
    for _ in range(3):
        y = y * (1.5 - 0.5 * v * y * y)
    return y


_mesh = plsc.VectorSubcoreMesh(
    core_axis_name="c", subcore_axis_name="s", num_cores=NC, num_subcores=NS
)


_SCRATCH = [
    pltpu.VMEM((NBUF, CHUNK, L), jnp.int32),
    pltpu.VMEM((NBUF, CHUNK, L, E), jnp.float32),
    pltpu.VMEM((CHUNK, E), jnp.float32),
    pltpu.VMEM((E,), jnp.float32),
    pltpu.VMEM((E,), jnp.float32),
    pltpu.SemaphoreType.DMA,
    pltpu.SemaphoreType.DMA,
    pltpu.SemaphoreType.DMA,
]


def _embed_body(idx_hbm, table_hbm, gamma_hbm, beta_hbm, out_hbm,
                  idx_v, rows_v, out_v, gamma_v, beta_v,
                  isem, gsem, osem):
    wid = lax.axis_index("s") * NC + lax.axis_index("c")
    base = wid * ROWS_PER_W

    pltpu.sync_copy(gamma_hbm, gamma_v)
    pltpu.sync_copy(beta_hbm, beta_v)

    def fire(t, buf):
        row0 = base + t * CHUNK
        pltpu.async_copy(idx_hbm.at[pl.ds(row0, CHUNK)], idx_v.at[buf], isem).wait()
        for j in range(CHUNK):
            pltpu.async_copy(table_hbm.at[idx_v.at[buf, j]], rows_v.at[buf, j], gsem)

    def drain(buf):
        for j in range(CHUNK):
            pltpu.make_async_copy(table_hbm.at[idx_v.at[buf, j]],
                                  rows_v.at[buf, j], gsem).wait()

    def compute(t, buf):
        row0 = base + t * CHUNK
        for j in range(CHUNK):
            def body(l, acc):
                return tuple(acc[c] + rows_v[buf, j, l, pl.ds(c * LANES, LANES)]
                             for c in range(EC))
            acc = lax.fori_loop(
                0, L, body,
                tuple(jnp.zeros((LANES,), jnp.float32) for _ in range(EC)),
            )
            m = [acc[c] * (1.0 / L) for c in range(EC)]
            tot = m[0] + m[1] + m[2] + m[3]
            mu = _lane_sum(tot) * (1.0 / E)
            d = [m[c] - mu for c in range(EC)]
            ss = d[0] * d[0] + d[1] * d[1] + d[2] * d[2] + d[3] * d[3]
            var = _lane_sum(ss) * (1.0 / E) + 1e-5
            inv = _rsqrt(var)
            for c in range(EC):
                sl = pl.ds(c * LANES, LANES)
                out_v[j, sl] = d[c] * inv * gamma_v[sl] + beta_v[sl]
        pltpu.async_copy(out_v, out_hbm.at[pl.ds(row0, CHUNK)], osem).wait()

    fire(0, 0)
    def step(t, carry):
        buf = lax.rem(t, NBUF)
        nbuf = lax.rem(t + 1, NBUF)

        @pl.when(t + 1 < NSTEPS)
        def _():
            fire(t + 1, nbuf)

        drain(buf)
        compute(t, buf)
        return carry

    lax.fori_loop(0, NSTEPS, step, 0)


_embed_kernel = functools.partial(
    pl.kernel,
    out_type=jax.ShapeDtypeStruct((B, E), jnp.float32),
    mesh=_mesh,
    scratch_types=_SCRATCH,
    compiler_params=pltpu.CompilerParams(use_tc_tiling_on_sc=False),
)(_embed_body)


def kernel(input, table, gamma, beta):
    idx = input.astype(jnp.int32)
    return _embed_kernel(idx, table, gamma, beta)


# speedup vs baseline: 61.1876x; 1.4291x over previous
import functools

import jax
import jax.numpy as jnp
from jax import lax
from jax.experimental import pallas as pl
from jax.experimental.pallas import tpu as pltpu
from jax.experimental.pallas import tpu_sc as plsc

NC, NS, LANES = 2, 16, 16
NW = NC * NS
B, L, E = 16384, 100, 64
EC = E // LANES
ROWS_PER_W = B // NW
CHUNK = 128
NSTEPS = ROWS_PER_W // CHUNK
NBUF = 2


def _lane_sum(x):
    lanes = lax.iota(jnp.int32, LANES)
    dnums = lax.GatherDimensionNumbers(
        offset_dims=(), collapsed_slice_dims=(0,), start_index_map=(0,)
    )
    for sh in (1, 2, 4, 8):
        perm = lax.reshape(lanes ^ sh, (LANES, 1))
        x = x + lax.gather(x, perm, dnums, slice_sizes=(1,),
                           mode=lax.GatherScatterMode.PROMISE_IN_BOUNDS)
    return x


def _rsqrt(v):
    i = lax.bitcast_convert_type(v, jnp.int32)
    i = 0x5F3759DF - lax.shift_right_logical(i, 1)
    y = lax.bitcast_convert_type(i, jnp.float32)
    for _ in range(3):
        y = y * (1.5 - 0.5 * v * y * y)
    return y


_mesh = plsc.VectorSubcoreMesh(
    core_axis_name="c", subcore_axis_name="s", num_cores=NC, num_subcores=NS
)

_SCRATCH = [
    pltpu.VMEM((NBUF, L, CHUNK), jnp.int32),
    pltpu.VMEM((NBUF, CHUNK, E), jnp.float32),
    pltpu.VMEM((CHUNK, E), jnp.float32),
    pltpu.VMEM((E,), jnp.float32),
    pltpu.VMEM((E,), jnp.float32),
    pltpu.SemaphoreType.DMA,
    pltpu.SemaphoreType.DMA,
    pltpu.SemaphoreType.DMA,
    pltpu.SemaphoreType.DMA,
]


def _embed_body(idx_hbm, table_hbm, gamma_hbm, beta_hbm, out_hbm,
                idx_v, acc_v, out_v, gamma_v, beta_v,
                isem, gsem0, gsem1, osem):
    wid = lax.axis_index("s") * NC + lax.axis_index("c")
    base = wid * ROWS_PER_W

    pltpu.sync_copy(gamma_hbm, gamma_v)
    pltpu.sync_copy(beta_hbm, beta_v)

    zeros = jnp.zeros((LANES,), jnp.float32)

    def zero_acc(buf):
        def zbody(j, carry):
            for c in range(EC):
                acc_v[buf, j, pl.ds(c * LANES, LANES)] = zeros
            return carry
        lax.fori_loop(0, CHUNK, zbody, 0)

    def fire(t, buf, gsem):
        row0 = base + t * CHUNK
        pltpu.async_copy(idx_hbm.at[:, pl.ds(row0, CHUNK)], idx_v.at[buf],
                         isem).wait()

        def gbody(l, carry):
            pltpu.async_copy(table_hbm.at[idx_v.at[buf, l]], acc_v.at[buf],
                             gsem, add=True)
            return carry
        lax.fori_loop(0, L, gbody, 0)

    def drain(buf, gsem):
        def wbody(l, carry):
            pltpu.make_async_copy(table_hbm.at[idx_v.at[buf, l]],
                                  acc_v.at[buf], gsem).wait()
            return carry
        lax.fori_loop(0, L, wbody, 0)

    def compute(t, buf):
        row0 = base + t * CHUNK

        def cbody(j, carry):
            m = [acc_v[buf, j, pl.ds(c * LANES, LANES)] * (1.0 / L)
                 for c in range(EC)]
            tot = m[0] + m[1] + m[2] + m[3]
            mu = _lane_sum(tot) * (1.0 / E)
            d = [m[c] - mu for c in range(EC)]
            ss = d[0] * d[0] + d[1] * d[1] + d[2] * d[2] + d[3] * d[3]
            var = _lane_sum(ss) * (1.0 / E) + 1e-5
            inv = _rsqrt(var)
            for c in range(EC):
                sl = pl.ds(c * LANES, LANES)
                out_v[j, sl] = d[c] * inv * gamma_v[sl] + beta_v[sl]
            return carry
        lax.fori_loop(0, CHUNK, cbody, 0)
        pltpu.async_copy(out_v, out_hbm.at[pl.ds(row0, CHUNK)], osem).wait()

    zero_acc(0)
    zero_acc(1)
    fire(0, 0, gsem0)

    def step(t, carry):
        buf = lax.rem(t, NBUF)

        @pl.when(t + 1 < NSTEPS)
        def _():
            @pl.when(lax.rem(t + 1, NBUF) == 0)
            def _():
                fire(t + 1, 0, gsem0)

            @pl.when(lax.rem(t + 1, NBUF) == 1)
            def _():
                fire(t + 1, 1, gsem1)

        @pl.when(buf == 0)
        def _():
            drain(0, gsem0)

        @pl.when(buf == 1)
        def _():
            drain(1, gsem1)

        compute(t, buf)
        zero_acc(buf)
        return carry

    lax.fori_loop(0, NSTEPS, step, 0)


_embed_kernel = functools.partial(
    pl.kernel,
    out_type=jax.ShapeDtypeStruct((B, E), jnp.float32),
    mesh=_mesh,
    scratch_types=_SCRATCH,
    compiler_params=pltpu.CompilerParams(use_tc_tiling_on_sc=False),
)(_embed_body)


def kernel(input, table, gamma, beta):
    idx_t = input.astype(jnp.int32).T
    return _embed_kernel(idx_t, table, gamma, beta)
